# baseline (speedup 1.0000x reference)
import functools

import jax
import jax.numpy as jnp
from jax import lax
from jax.experimental import pallas as pl
from jax.experimental.pallas import tpu as pltpu
from jax.experimental.pallas import tpu_sc as plsc

_N_REL = 3
_D = 128

_NC = 2
_NS = 16
_NW = _NC * _NS
_BLK = 128
_NBLK = 80
_TOTBLK = _NW * _NBLK
_NBLK0 = 48
_NBLK1 = 2 * _NBLK - _NBLK0
_EPAD = _TOTBLK * _BLK


def _sc_aggregate(meta, embs):
    n_rel = meta.shape[0]
    n_pad = ((embs.shape[0] + 639) // 640) * 640
    rows_per_tile = n_pad // _NS

    mesh = plsc.VectorSubcoreMesh(core_axis_name="c", subcore_axis_name="s")

    @functools.partial(
        pl.kernel,
        mesh=mesh,
        compiler_params=pltpu.CompilerParams(needs_layout_passes=False),
        out_type=jax.ShapeDtypeStruct((n_rel, _NC, n_pad, _D), jnp.float32),
        scratch_types=[
            pltpu.VMEM((3, 128), jnp.int32),
            pltpu.VMEM((3, 128), jnp.int32),
            pltpu.VMEM((_BLK, _D), jnp.float32),
            pltpu.VMEM((_BLK, _D), jnp.float32),
            pltpu.VMEM_SHARED((n_pad, _D), jnp.float32),
            pltpu.SemaphoreType.DMA,
            pltpu.SemaphoreType.DMA,
            pltpu.SemaphoreType.DMA,
            pltpu.SemaphoreType.DMA,
        ],
    )
    def k(meta_hbm, embs_hbm, out_hbm,
          meta0, meta1, rows0, rows1, acc, gsem0, gsem1, ssem0, ssem1):
        cid = lax.axis_index("c")
        sid = lax.axis_index("s")
        row0 = sid * rows_per_tile
        blk0 = jnp.where(cid == 0, sid * _NBLK0, _NS * _NBLK0 + sid * _NBLK1)
        n_pairs = jnp.where(cid == 0, _NBLK0 // 2, _NBLK1 // 2)
        nblk = jnp.where(cid == 0, _NBLK0, _NBLK1)

        zero16 = jnp.zeros((16,), jnp.float32)
        two16 = jnp.full((16,), 2, jnp.int32)

        def zero_row(i, c):
            for kk in range(_D // 16):
                rows0[i, pl.ds(kk * 16, 16)] = zero16
            return c

        def scale(rows_ref, meta_ref):
            def grp(gi, c):
                for j in range(16):
                    col = gi * 16 + j
                    vbits = plsc.load_gather(
                        meta_ref, [two16, jnp.full((16,), col, jnp.int32)])
                    vspl = plsc.bitcast(vbits, jnp.float32)
                    for kk in range(_D // 16):
                        rows_ref[col, pl.ds(kk * 16, 16)] = (
                            rows_ref[col, pl.ds(kk * 16, 16)] * vspl)
                return c

            lax.fori_loop(0, _BLK // 16, grp, 0)

        for r in range(n_rel):
            lax.fori_loop(0, _BLK, zero_row, 0)
            for j in range(rows_per_tile // _BLK):
                pltpu.sync_copy(rows0, acc.at[pl.ds(row0 + j * _BLK, _BLK)])
            plsc.subcore_barrier()

            pltpu.sync_copy(meta_hbm.at[r, blk0], meta0)
            pltpu.async_copy(embs_hbm.at[meta0.at[0]], rows0, gsem0)

            def pair(g, c, r=r):
                b0 = 2 * g
                pltpu.make_async_copy(
                    embs_hbm.at[meta0.at[0]], rows0, gsem0).wait()
                scale(rows0, meta0)

                @pl.when(g > 0)
                def _():
                    pltpu.make_async_copy(
                        rows1, acc.at[meta1.at[1]], ssem1).wait()

                pltpu.sync_copy(meta_hbm.at[r, blk0 + b0 + 1], meta1)
                g1 = pltpu.async_copy(embs_hbm.at[meta1.at[0]], rows1, gsem1)
                s0 = pltpu.async_copy(rows0, acc.at[meta0.at[1]], ssem0,
                                      add=True)
                g1.wait()
                scale(rows1, meta1)
                s0.wait()

                @pl.when(b0 + 2 < nblk)
                def _():
                    pltpu.sync_copy(meta_hbm.at[r, blk0 + b0 + 2], meta0)
                    pltpu.async_copy(embs_hbm.at[meta0.at[0]], rows0, gsem0)

                pltpu.async_copy(rows1, acc.at[meta1.at[1]], ssem1, add=True)
                return c

            lax.fori_loop(0, n_pairs, pair, 0)
            pltpu.make_async_copy(rows1, acc.at[meta1.at[1]], ssem1).wait()
            plsc.subcore_barrier()
            pltpu.sync_copy(acc.at[pl.ds(row0, rows_per_tile)],
                            out_hbm.at[r, cid, pl.ds(row0, rows_per_tile)])

    return k(meta, embs)


def _tc_combine(partials, n_nodes):
    n_rel, nc, n_pad, d = partials.shape
    p = partials.reshape(n_rel * nc, n_pad, d)
    br = 1000

    def body(p_ref, o_ref):
        acc = None
        for r in range(n_rel):
            x = p_ref[nc * r]
            for c in range(1, nc):
                x = x + p_ref[nc * r + c]
            y = jnp.maximum(x, 0.01 * x)
            acc = y if acc is None else acc + y
        o_ref[...] = acc

    return pl.pallas_call(
        body,
        grid=(n_nodes // br,),
        in_specs=[pl.BlockSpec((n_rel * nc, br, d), lambda i: (0, i, 0))],
        out_specs=pl.BlockSpec((br, d), lambda i: (i, 0)),
        out_shape=jax.ShapeDtypeStruct((n_nodes, d), jnp.float32),
    )(p)


def kernel(embs, edge_index, edge_vals):
    dst = edge_index[:, 0, :].astype(jnp.int32)
    src = edge_index[:, 1, :].astype(jnp.int32)
    vals = edge_vals.astype(jnp.float32)
    pad = _EPAD - src.shape[1]
    src = jnp.pad(src, ((0, 0), (0, pad))).reshape(_N_REL, _TOTBLK, 128)
    dst = jnp.pad(dst, ((0, 0), (0, pad))).reshape(_N_REL, _TOTBLK, 128)
    vbits = lax.bitcast_convert_type(
        jnp.pad(vals, ((0, 0), (0, pad))), jnp.int32
    ).reshape(_N_REL, _TOTBLK, 128)
    meta = jnp.stack([src, dst, vbits], axis=2)
    partials = _sc_aggregate(meta, embs)
    return _tc_combine(partials, embs.shape[0])


# speedup vs baseline: 1.2998x; 1.2998x over previous
import functools

import jax
import jax.numpy as jnp
from jax import lax
from jax.experimental import pallas as pl
from jax.experimental.pallas import tpu as pltpu
from jax.experimental.pallas import tpu_sc as plsc

_N_REL = 3
_D = 128

_NC = 2
_NS = 16
_NW = _NC * _NS
_BLK = 128
_NBLK = 80
_TOTBLK = _NW * _NBLK
_NBLK0 = 106
_NBLK1 = 2 * _NBLK - _NBLK0
_EPAD = _TOTBLK * _BLK


def _sc_aggregate(meta, embs):
    n_rel = meta.shape[0]
    n_pad = ((embs.shape[0] + 639) // 640) * 640
    rows_per_tile = n_pad // _NS

    mesh = plsc.VectorSubcoreMesh(core_axis_name="c", subcore_axis_name="s")

    @functools.partial(
        pl.kernel,
        mesh=mesh,
        compiler_params=pltpu.CompilerParams(needs_layout_passes=False),
        out_type=jax.ShapeDtypeStruct((n_rel, _NC, n_pad, _D), jnp.float32),
        scratch_types=[
            pltpu.VMEM((3, 128), jnp.int32),
            pltpu.VMEM((3, 128), jnp.int32),
            pltpu.VMEM((_BLK, _D), jnp.float32),
            pltpu.VMEM((_BLK, _D), jnp.float32),
            pltpu.VMEM_SHARED((n_pad, _D), jnp.float32),
            pltpu.SemaphoreType.DMA,
            pltpu.SemaphoreType.DMA,
            pltpu.SemaphoreType.DMA,
            pltpu.SemaphoreType.DMA,
        ],
    )
    def k(meta_hbm, embs_hbm, out_hbm,
          meta0, meta1, rows0, rows1, acc, gsem0, gsem1, ssem0, ssem1):
        cid = lax.axis_index("c")
        sid = lax.axis_index("s")
        row0 = sid * rows_per_tile
        blk0 = jnp.where(cid == 0, sid * _NBLK0, _NS * _NBLK0 + sid * _NBLK1)
        n_pairs = jnp.where(cid == 0, _NBLK0 // 2, _NBLK1 // 2)
        nblk = jnp.where(cid == 0, _NBLK0, _NBLK1)

        zero16 = jnp.zeros((16,), jnp.float32)
        two16 = jnp.full((16,), 2, jnp.int32)

        def zero_row(i, c):
            for kk in range(_D // 16):
                rows0[i, pl.ds(kk * 16, 16)] = zero16
            return c

        def scale(rows_ref, meta_ref):
            def grp(gi, c):
                for j in range(16):
                    col = gi * 16 + j
                    vbits = plsc.load_gather(
                        meta_ref, [two16, jnp.full((16,), col, jnp.int32)])
                    vspl = plsc.bitcast(vbits, jnp.float32)
                    for kk in range(_D // 16):
                        rows_ref[col, pl.ds(kk * 16, 16)] = (
                            rows_ref[col, pl.ds(kk * 16, 16)] * vspl)
                return c

            lax.fori_loop(0, _BLK // 16, grp, 0)

        for r in range(n_rel):
            lax.fori_loop(0, _BLK, zero_row, 0)
            for j in range(rows_per_tile // _BLK):
                pltpu.sync_copy(rows0, acc.at[pl.ds(row0 + j * _BLK, _BLK)])
            plsc.subcore_barrier()

            pltpu.sync_copy(meta_hbm.at[r, blk0], meta0)
            pltpu.async_copy(embs_hbm.at[meta0.at[0]], rows0, gsem0)

            def pair(g, c, r=r):
                b0 = 2 * g
                pltpu.make_async_copy(
                    embs_hbm.at[meta0.at[0]], rows0, gsem0).wait()
                scale(rows0, meta0)

                @pl.when(g > 0)
                def _():
                    pltpu.make_async_copy(
                        rows1, acc.at[meta1.at[1]], ssem1).wait()

                pltpu.sync_copy(meta_hbm.at[r, blk0 + b0 + 1], meta1)
                g1 = pltpu.async_copy(embs_hbm.at[meta1.at[0]], rows1, gsem1)
                s0 = pltpu.async_copy(rows0, acc.at[meta0.at[1]], ssem0,
                                      add=True)
                g1.wait()
                scale(rows1, meta1)
                s0.wait()

                @pl.when(b0 + 2 < nblk)
                def _():
                    pltpu.sync_copy(meta_hbm.at[r, blk0 + b0 + 2], meta0)
                    pltpu.async_copy(embs_hbm.at[meta0.at[0]], rows0, gsem0)

                pltpu.async_copy(rows1, acc.at[meta1.at[1]], ssem1, add=True)
                return c

            lax.fori_loop(0, n_pairs, pair, 0)
            pltpu.make_async_copy(rows1, acc.at[meta1.at[1]], ssem1).wait()
            plsc.subcore_barrier()
            pltpu.sync_copy(acc.at[pl.ds(row0, rows_per_tile)],
                            out_hbm.at[r, cid, pl.ds(row0, rows_per_tile)])

    return k(meta, embs)


def _tc_combine(partials, n_nodes):
    n_rel, nc, n_pad, d = partials.shape
    p = partials.reshape(n_rel * nc, n_pad, d)
    br = 1000

    def body(p_ref, o_ref):
        acc = None
        for r in range(n_rel):
            x = p_ref[nc * r]
            for c in range(1, nc):
                x = x + p_ref[nc * r + c]
            y = jnp.maximum(x, 0.01 * x)
            acc = y if acc is None else acc + y
        o_ref[...] = acc

    return pl.pallas_call(
        body,
        grid=(n_nodes // br,),
        in_specs=[pl.BlockSpec((n_rel * nc, br, d), lambda i: (0, i, 0))],
        out_specs=pl.BlockSpec((br, d), lambda i: (i, 0)),
        out_shape=jax.ShapeDtypeStruct((n_nodes, d), jnp.float32),
    )(p)


def kernel(embs, edge_index, edge_vals):
    dst = edge_index[:, 0, :].astype(jnp.int32)
    src = edge_index[:, 1, :].astype(jnp.int32)
    vals = edge_vals.astype(jnp.float32)
    pad = _EPAD - src.shape[1]
    src = jnp.pad(src, ((0, 0), (0, pad))).reshape(_N_REL, _TOTBLK, 128)
    dst = jnp.pad(dst, ((0, 0), (0, pad))).reshape(_N_REL, _TOTBLK, 128)
    vbits = lax.bitcast_convert_type(
        jnp.pad(vals, ((0, 0), (0, pad))), jnp.int32
    ).reshape(_N_REL, _TOTBLK, 128)
    meta = jnp.stack([src, dst, vbits], axis=2)
    partials = _sc_aggregate(meta, embs)
    return _tc_combine(partials, embs.shape[0])


# speedup vs baseline: 1.5587x; 1.1993x over previous
import functools

import jax
import jax.numpy as jnp
from jax import lax
from jax.experimental import pallas as pl
from jax.experimental.pallas import tpu as pltpu
from jax.experimental.pallas import tpu_sc as plsc

_N_REL = 3
_D = 128

_NC = 2
_NS = 16
_NW = _NC * _NS
_BLK = 128
_NBLK = 80
_TOTBLK = _NW * _NBLK
_NBLK0 = 136
_NBLK1 = 2 * _NBLK - _NBLK0
_EPAD = _TOTBLK * _BLK


def _sc_aggregate(meta, embs):
    n_rel = meta.shape[0]
    n_pad = ((embs.shape[0] + 639) // 640) * 640
    rows_per_tile = n_pad // _NS

    mesh = plsc.VectorSubcoreMesh(core_axis_name="c", subcore_axis_name="s")

    @functools.partial(
        pl.kernel,
        mesh=mesh,
        compiler_params=pltpu.CompilerParams(needs_layout_passes=False),
        out_type=jax.ShapeDtypeStruct((n_rel, _NC, n_pad, _D), jnp.float32),
        scratch_types=[
            pltpu.VMEM((3, 128), jnp.int32),
            pltpu.VMEM((3, 128), jnp.int32),
            pltpu.VMEM((_BLK, _D), jnp.float32),
            pltpu.VMEM((_BLK, _D), jnp.float32),
            pltpu.VMEM_SHARED((n_pad, _D), jnp.float32),
            pltpu.SemaphoreType.DMA,
            pltpu.SemaphoreType.DMA,
            pltpu.SemaphoreType.DMA,
            pltpu.SemaphoreType.DMA,
        ],
    )
    def k(meta_hbm, embs_hbm, out_hbm,
          meta0, meta1, rows0, rows1, acc, gsem0, gsem1, ssem0, ssem1):
        cid = lax.axis_index("c")
        sid = lax.axis_index("s")
        row0 = sid * rows_per_tile
        blk0 = jnp.where(cid == 0, sid * _NBLK0, _NS * _NBLK0 + sid * _NBLK1)
        n_pairs = jnp.where(cid == 0, _NBLK0 // 2, _NBLK1 // 2)
        nblk = jnp.where(cid == 0, _NBLK0, _NBLK1)

        zero16 = jnp.zeros((16,), jnp.float32)
        two16 = jnp.full((16,), 2, jnp.int32)

        def zero_row(i, c):
            for kk in range(_D // 16):
                rows0[i, pl.ds(kk * 16, 16)] = zero16
            return c

        def scale(rows_ref, meta_ref):
            def grp(gi, c):
                for j in range(16):
                    col = gi * 16 + j
                    vbits = plsc.load_gather(
                        meta_ref, [two16, jnp.full((16,), col, jnp.int32)])
                    vspl = plsc.bitcast(vbits, jnp.float32)
                    for kk in range(_D // 16):
                        rows_ref[col, pl.ds(kk * 16, 16)] = (
                            rows_ref[col, pl.ds(kk * 16, 16)] * vspl)
                return c

            lax.fori_loop(0, _BLK // 16, grp, 0)

        for r in range(n_rel):
            lax.fori_loop(0, _BLK, zero_row, 0)
            for j in range(rows_per_tile // _BLK):
                pltpu.sync_copy(rows0, acc.at[pl.ds(row0 + j * _BLK, _BLK)])
            plsc.subcore_barrier()

            pltpu.sync_copy(meta_hbm.at[r, blk0], meta0)
            pltpu.async_copy(embs_hbm.at[meta0.at[0]], rows0, gsem0)

            def pair(g, c, r=r):
                b0 = 2 * g
                pltpu.make_async_copy(
                    embs_hbm.at[meta0.at[0]], rows0, gsem0).wait()
                scale(rows0, meta0)

                @pl.when(g > 0)
                def _():
                    pltpu.make_async_copy(
                        rows1, acc.at[meta1.at[1]], ssem1).wait()

                pltpu.sync_copy(meta_hbm.at[r, blk0 + b0 + 1], meta1)
                g1 = pltpu.async_copy(embs_hbm.at[meta1.at[0]], rows1, gsem1)
                s0 = pltpu.async_copy(rows0, acc.at[meta0.at[1]], ssem0,
                                      add=True)
                g1.wait()
                scale(rows1, meta1)
                s0.wait()

                @pl.when(b0 + 2 < nblk)
                def _():
                    pltpu.sync_copy(meta_hbm.at[r, blk0 + b0 + 2], meta0)
                    pltpu.async_copy(embs_hbm.at[meta0.at[0]], rows0, gsem0)

                pltpu.async_copy(rows1, acc.at[meta1.at[1]], ssem1, add=True)
                return c

            lax.fori_loop(0, n_pairs, pair, 0)
            pltpu.make_async_copy(rows1, acc.at[meta1.at[1]], ssem1).wait()
            plsc.subcore_barrier()
            pltpu.sync_copy(acc.at[pl.ds(row0, rows_per_tile)],
                            out_hbm.at[r, cid, pl.ds(row0, rows_per_tile)])

    return k(meta, embs)


def _tc_combine(partials, n_nodes):
    n_rel, nc, n_pad, d = partials.shape
    p = partials.reshape(n_rel * nc, n_pad, d)
    br = 1000

    def body(p_ref, o_ref):
        acc = None
        for r in range(n_rel):
            x = p_ref[nc * r]
            for c in range(1, nc):
                x = x + p_ref[nc * r + c]
            y = jnp.maximum(x, 0.01 * x)
            acc = y if acc is None else acc + y
        o_ref[...] = acc

    return pl.pallas_call(
        body,
        grid=(n_nodes // br,),
        in_specs=[pl.BlockSpec((n_rel * nc, br, d), lambda i: (0, i, 0))],
        out_specs=pl.BlockSpec((br, d), lambda i: (i, 0)),
        out_shape=jax.ShapeDtypeStruct((n_nodes, d), jnp.float32),
    )(p)


def kernel(embs, edge_index, edge_vals):
    dst = edge_index[:, 0, :].astype(jnp.int32)
    src = edge_index[:, 1, :].astype(jnp.int32)
    vals = edge_vals.astype(jnp.float32)
    pad = _EPAD - src.shape[1]
    src = jnp.pad(src, ((0, 0), (0, pad))).reshape(_N_REL, _TOTBLK, 128)
    dst = jnp.pad(dst, ((0, 0), (0, pad))).reshape(_N_REL, _TOTBLK, 128)
    vbits = lax.bitcast_convert_type(
        jnp.pad(vals, ((0, 0), (0, pad))), jnp.int32
    ).reshape(_N_REL, _TOTBLK, 128)
    meta = jnp.stack([src, dst, vbits], axis=2)
    partials = _sc_aggregate(meta, embs)
    return _tc_combine(partials, embs.shape[0])


# speedup vs baseline: 1.7511x; 1.1234x over previous
import functools

import jax
import jax.numpy as jnp
from jax import lax
from jax.experimental import pallas as pl
from jax.experimental.pallas import tpu as pltpu
from jax.experimental.pallas import tpu_sc as plsc

_N_REL = 3
_D = 128

_NC = 2
_NS = 16
_NW = _NC * _NS
_BLK = 128
_NBLK = 80
_TOTBLK = _NW * _NBLK
_NBLK0 = 136
_NBLK1 = 2 * _NBLK - _NBLK0
_EPAD = _TOTBLK * _BLK


def _sc_aggregate(meta, embs):
    n_rel = meta.shape[0]
    n_pad = ((embs.shape[0] + 639) // 640) * 640
    rows_per_tile = n_pad // _NS

    mesh = plsc.VectorSubcoreMesh(core_axis_name="c", subcore_axis_name="s")

    @functools.partial(
        pl.kernel,
        mesh=mesh,
        compiler_params=pltpu.CompilerParams(needs_layout_passes=False),
        out_type=jax.ShapeDtypeStruct((n_rel, _NC, n_pad, _D), jnp.float32),
        scratch_types=[
            pltpu.VMEM((6, 128), jnp.int32),
            pltpu.VMEM((6, 128), jnp.int32),
            pltpu.VMEM((_BLK, _D), jnp.float32),
            pltpu.VMEM((_BLK, _D), jnp.float32),
            pltpu.VMEM((1, 128), jnp.int32),
            pltpu.VMEM((1, 128), jnp.int32),
            pltpu.VMEM_SHARED((n_pad, _D), jnp.float32),
            pltpu.SemaphoreType.DMA,
            pltpu.SemaphoreType.DMA,
            pltpu.SemaphoreType.DMA,
            pltpu.SemaphoreType.DMA,
        ],
    )
    def k(meta_hbm, embs_hbm, out_hbm,
          mpA, mpB, rows0, rows1, sx0, sx1, acc,
          gsem0, gsem1, ssem0, ssem1):
        cid = lax.axis_index("c")
        sid = lax.axis_index("s")
        row0 = sid * rows_per_tile
        blk0 = jnp.where(cid == 0, sid * _NBLK0, _NS * _NBLK0 + sid * _NBLK1)
        n_quads = jnp.where(cid == 0, _NBLK0 // 4, _NBLK1 // 4)

        zero16 = jnp.zeros((16,), jnp.float32)
        rows = [rows0, rows1]
        sxs = [sx0, sx1]
        gsems = [gsem0, gsem1]
        ssems = [ssem0, ssem1]

        def zero_row(i, c):
            for kk in range(_D // 16):
                rows0[i, pl.ds(kk * 16, 16)] = zero16
            return c

        def fire_g(mp, half, h):
            pltpu.async_copy(embs_hbm.at[mp.at[3 * half]], rows[h], gsems[h])

        def drain_g(mp, half, h):
            pltpu.make_async_copy(
                embs_hbm.at[mp.at[3 * half]], rows[h], gsems[h]).wait()

        def fire_s(h):
            pltpu.async_copy(rows[h], acc.at[sxs[h].at[0]], ssems[h],
                             add=True)

        def drain_s(h):
            pltpu.make_async_copy(
                rows[h], acc.at[sxs[h].at[0]], ssems[h]).wait()

        def scale(mp, half, h):
            vrow = jnp.full((16,), 3 * half + 2, jnp.int32)

            def grp(gi, c):
                for j in range(16):
                    col = gi * 16 + j
                    vbits = plsc.load_gather(
                        mp, [vrow, jnp.full((16,), col, jnp.int32)])
                    vspl = plsc.bitcast(vbits, jnp.float32)
                    for kk in range(_D // 16):
                        rows[h][col, pl.ds(kk * 16, 16)] = (
                            rows[h][col, pl.ds(kk * 16, 16)] * vspl)
                return c

            lax.fori_loop(0, _BLK // 16, grp, 0)
            for kk in range(_D // 16):
                sxs[h][0, pl.ds(kk * 16, 16)] = (
                    mp[3 * half + 1, pl.ds(kk * 16, 16)])

        for r in range(n_rel):
            lax.fori_loop(0, _BLK, zero_row, 0)
            for j in range(rows_per_tile // _BLK):
                pltpu.sync_copy(rows0, acc.at[pl.ds(row0 + j * _BLK, _BLK)])
            plsc.subcore_barrier()

            pblk0 = blk0 // 2
            pltpu.sync_copy(meta_hbm.at[r, pblk0], mpA)
            fire_g(mpA, 0, 0)
            fire_g(mpA, 1, 1)

            def quad(q, c, r=r):
                drain_g(mpA, 0, 0)
                scale(mpA, 0, 0)
                fire_s(0)
                drain_g(mpA, 1, 1)
                scale(mpA, 1, 1)
                fire_s(1)
                pltpu.sync_copy(meta_hbm.at[r, pblk0 + 2 * q + 1], mpB)
                drain_s(0)
                fire_g(mpB, 0, 0)
                drain_s(1)
                fire_g(mpB, 1, 1)
                drain_g(mpB, 0, 0)
                scale(mpB, 0, 0)
                fire_s(0)
                drain_g(mpB, 1, 1)
                scale(mpB, 1, 1)
                fire_s(1)

                @pl.when(q + 1 < n_quads)
                def _(r=r):
                    pltpu.sync_copy(meta_hbm.at[r, pblk0 + 2 * q + 2], mpA)

                drain_s(0)

                @pl.when(q + 1 < n_quads)
                def _():
                    fire_g(mpA, 0, 0)

                drain_s(1)

                @pl.when(q + 1 < n_quads)
                def _():
                    fire_g(mpA, 1, 1)

                return c

            lax.fori_loop(0, n_quads, quad, 0)
            plsc.subcore_barrier()
            pltpu.sync_copy(acc.at[pl.ds(row0, rows_per_tile)],
                            out_hbm.at[r, cid, pl.ds(row0, rows_per_tile)])

    return k(meta, embs)


def _tc_combine(partials, n_nodes):
    n_rel, nc, n_pad, d = partials.shape
    p = partials.reshape(n_rel * nc, n_pad, d)
    br = 1000

    def body(p_ref, o_ref):
        acc = None
        for r in range(n_rel):
            x = p_ref[nc * r]
            for c in range(1, nc):
                x = x + p_ref[nc * r + c]
            y = jnp.maximum(x, 0.01 * x)
            acc = y if acc is None else acc + y
        o_ref[...] = acc

    return pl.pallas_call(
        body,
        grid=(n_nodes // br,),
        in_specs=[pl.BlockSpec((n_rel * nc, br, d), lambda i: (0, i, 0))],
        out_specs=pl.BlockSpec((br, d), lambda i: (i, 0)),
        out_shape=jax.ShapeDtypeStruct((n_nodes, d), jnp.float32),
    )(p)


def kernel(embs, edge_index, edge_vals):
    n = embs.shape[0]
    dst = edge_index[:, 0, :].astype(jnp.int32)
    src = edge_index[:, 1, :].astype(jnp.int32)
    vals = edge_vals.astype(jnp.float32)
    pad = _EPAD - src.shape[1]
    pad_dst = jnp.broadcast_to(
        jnp.arange(pad, dtype=jnp.int32) % n, (_N_REL, pad))
    src = jnp.pad(src, ((0, 0), (0, pad))).reshape(_N_REL, _TOTBLK, 128)
    dst = jnp.concatenate([dst, pad_dst], axis=1).reshape(_N_REL, _TOTBLK, 128)
    vbits = lax.bitcast_convert_type(
        jnp.pad(vals, ((0, 0), (0, pad))), jnp.int32
    ).reshape(_N_REL, _TOTBLK, 128)
    meta = jnp.stack([src, dst, vbits], axis=2).reshape(
        _N_REL, _TOTBLK // 2, 6, 128)
    partials = _sc_aggregate(meta, embs)
    return _tc_combine(partials, embs.shape[0])


# speedup vs baseline: 1.8150x; 1.0365x over previous
import functools

import jax
import jax.numpy as jnp
from jax import lax
from jax.experimental import pallas as pl
from jax.experimental.pallas import tpu as pltpu
from jax.experimental.pallas import tpu_sc as plsc

_N_REL = 3
_D = 128

_NC = 2
_NS = 16
_NW = _NC * _NS
_BLK = 128
_NBLK = 80
_TOTBLK = _NW * _NBLK
_NBLK0 = 148
_NBLK1 = 2 * _NBLK - _NBLK0
_EPAD = _TOTBLK * _BLK


def _sc_aggregate(meta, embs):
    n_rel = meta.shape[0]
    n_pad = ((embs.shape[0] + 639) // 640) * 640
    rows_per_tile = n_pad // _NS

    mesh = plsc.VectorSubcoreMesh(core_axis_name="c", subcore_axis_name="s")

    @functools.partial(
        pl.kernel,
        mesh=mesh,
        compiler_params=pltpu.CompilerParams(needs_layout_passes=False),
        out_type=jax.ShapeDtypeStruct((n_rel, _NC, n_pad, _D), jnp.float32),
        scratch_types=[
            pltpu.VMEM((6, 128), jnp.int32),
            pltpu.VMEM((6, 128), jnp.int32),
            pltpu.VMEM((_BLK, _D), jnp.float32),
            pltpu.VMEM((_BLK, _D), jnp.float32),
            pltpu.VMEM((1, 128), jnp.int32),
            pltpu.VMEM((1, 128), jnp.int32),
            pltpu.VMEM_SHARED((n_pad, _D), jnp.float32),
            pltpu.SemaphoreType.DMA,
            pltpu.SemaphoreType.DMA,
            pltpu.SemaphoreType.DMA,
            pltpu.SemaphoreType.DMA,
        ],
    )
    def k(meta_hbm, embs_hbm, out_hbm,
          mpA, mpB, rows0, rows1, sx0, sx1, acc,
          gsem0, gsem1, ssem0, ssem1):
        cid = lax.axis_index("c")
        sid = lax.axis_index("s")
        row0 = sid * rows_per_tile
        blk0 = jnp.where(cid == 0, sid * _NBLK0, _NS * _NBLK0 + sid * _NBLK1)
        n_quads = jnp.where(cid == 0, _NBLK0 // 4, _NBLK1 // 4)

        zero16 = jnp.zeros((16,), jnp.float32)
        rows = [rows0, rows1]
        sxs = [sx0, sx1]
        gsems = [gsem0, gsem1]
        ssems = [ssem0, ssem1]

        def zero_row(i, c):
            for kk in range(_D // 16):
                rows0[i, pl.ds(kk * 16, 16)] = zero16
            return c

        def fire_g(mp, half, h):
            pltpu.async_copy(embs_hbm.at[mp.at[3 * half]], rows[h], gsems[h])

        def drain_g(mp, half, h):
            pltpu.make_async_copy(
                embs_hbm.at[mp.at[3 * half]], rows[h], gsems[h]).wait()

        def fire_s(h):
            pltpu.async_copy(rows[h], acc.at[sxs[h].at[0]], ssems[h],
                             add=True)

        def drain_s(h):
            pltpu.make_async_copy(
                rows[h], acc.at[sxs[h].at[0]], ssems[h]).wait()

        def scale(mp, half, h):
            vrow = jnp.full((16,), 3 * half + 2, jnp.int32)

            def grp(gi, c):
                for j in range(16):
                    col = gi * 16 + j
                    vbits = plsc.load_gather(
                        mp, [vrow, jnp.full((16,), col, jnp.int32)])
                    vspl = plsc.bitcast(vbits, jnp.float32)
                    for kk in range(_D // 16):
                        rows[h][col, pl.ds(kk * 16, 16)] = (
                            rows[h][col, pl.ds(kk * 16, 16)] * vspl)
                return c

            lax.fori_loop(0, _BLK // 16, grp, 0)
            for kk in range(_D // 16):
                sxs[h][0, pl.ds(kk * 16, 16)] = (
                    mp[3 * half + 1, pl.ds(kk * 16, 16)])

        for r in range(n_rel):
            lax.fori_loop(0, _BLK, zero_row, 0)
            for j in range(rows_per_tile // _BLK):
                pltpu.sync_copy(rows0, acc.at[pl.ds(row0 + j * _BLK, _BLK)])
            plsc.subcore_barrier()

            pblk0 = blk0 // 2
            pltpu.sync_copy(meta_hbm.at[r, pblk0], mpA)
            fire_g(mpA, 0, 0)
            fire_g(mpA, 1, 1)

            def quad(q, c, r=r):
                drain_g(mpA, 0, 0)
                scale(mpA, 0, 0)
                fire_s(0)
                drain_g(mpA, 1, 1)
                scale(mpA, 1, 1)
                fire_s(1)
                pltpu.sync_copy(meta_hbm.at[r, pblk0 + 2 * q + 1], mpB)
                drain_s(0)
                fire_g(mpB, 0, 0)
                drain_s(1)
                fire_g(mpB, 1, 1)
                drain_g(mpB, 0, 0)
                scale(mpB, 0, 0)
                fire_s(0)
                drain_g(mpB, 1, 1)
                scale(mpB, 1, 1)
                fire_s(1)

                @pl.when(q + 1 < n_quads)
                def _(r=r):
                    pltpu.sync_copy(meta_hbm.at[r, pblk0 + 2 * q + 2], mpA)

                drain_s(0)

                @pl.when(q + 1 < n_quads)
                def _():
                    fire_g(mpA, 0, 0)

                drain_s(1)

                @pl.when(q + 1 < n_quads)
                def _():
                    fire_g(mpA, 1, 1)

                return c

            lax.fori_loop(0, n_quads, quad, 0)
            plsc.subcore_barrier()
            pltpu.sync_copy(acc.at[pl.ds(row0, rows_per_tile)],
                            out_hbm.at[r, cid, pl.ds(row0, rows_per_tile)])

    return k(meta, embs)


def _tc_combine(partials, n_nodes):
    n_rel, nc, n_pad, d = partials.shape
    p = partials.reshape(n_rel * nc, n_pad, d)
    br = 1000

    def body(p_ref, o_ref):
        acc = None
        for r in range(n_rel):
            x = p_ref[nc * r]
            for c in range(1, nc):
                x = x + p_ref[nc * r + c]
            y = jnp.maximum(x, 0.01 * x)
            acc = y if acc is None else acc + y
        o_ref[...] = acc

    return pl.pallas_call(
        body,
        grid=(n_nodes // br,),
        in_specs=[pl.BlockSpec((n_rel * nc, br, d), lambda i: (0, i, 0))],
        out_specs=pl.BlockSpec((br, d), lambda i: (i, 0)),
        out_shape=jax.ShapeDtypeStruct((n_nodes, d), jnp.float32),
    )(p)


def kernel(embs, edge_index, edge_vals):
    n = embs.shape[0]
    dst = edge_index[:, 0, :].astype(jnp.int32)
    src = edge_index[:, 1, :].astype(jnp.int32)
    vals = edge_vals.astype(jnp.float32)
    pad = _EPAD - src.shape[1]
    pad_dst = jnp.broadcast_to(
        jnp.arange(pad, dtype=jnp.int32) % n, (_N_REL, pad))
    src = jnp.pad(src, ((0, 0), (0, pad))).reshape(_N_REL, _TOTBLK, 128)
    dst = jnp.concatenate([dst, pad_dst], axis=1).reshape(_N_REL, _TOTBLK, 128)
    vbits = lax.bitcast_convert_type(
        jnp.pad(vals, ((0, 0), (0, pad))), jnp.int32
    ).reshape(_N_REL, _TOTBLK, 128)
    meta = jnp.stack([src, dst, vbits], axis=2).reshape(
        _N_REL, _TOTBLK // 2, 6, 128)
    partials = _sc_aggregate(meta, embs)
    return _tc_combine(partials, embs.shape[0])
